# baseline (speedup 1.0000x reference)
import functools

import jax
import jax.numpy as jnp
from jax import lax
from jax.experimental import pallas as pl
from jax.experimental.pallas import tpu as pltpu
from jax.experimental.pallas import tpu_sc as plsc

N = 10000
E = 320000
D = 128
C = 64
L = 4

NCORE = 2
NSUB = 16
NP = 10240
RT = NP // NSUB
HF = 16
K = 128
NCHUNK = 157
EPAD = NSUB * NCHUNK * K
EW = E // (NCORE * NSUB)

_mesh = plsc.VectorSubcoreMesh(core_axis_name="c", subcore_axis_name="s")
_params = pltpu.CompilerParams(
    needs_layout_passes=False, use_tc_tiling_on_sc=False)


def _deg_body(row_hbm, out_hbm, idx_v, hist_v):
    cid = lax.axis_index("c")
    sid = lax.axis_index("s")
    wid = cid * NSUB + sid
    pltpu.sync_copy(row_hbm.at[wid], idx_v)

    def zero_body(i, carry):
        hist_v[pl.ds(i * 16, 16)] = jnp.zeros((16,), jnp.float32)
        return carry

    lax.fori_loop(0, NP // 16, zero_body, 0)

    ones16 = jnp.ones((16,), jnp.float32)

    def acc_body(j, carry):
        idx16 = idx_v[pl.ds(j * 16, 16)]
        plsc.addupdate_scatter(hist_v, [idx16], ones16)
        return carry

    lax.fori_loop(0, EW // 16, acc_body, 0)
    pltpu.sync_copy(hist_v, out_hbm.at[wid])


_deg_kernel = functools.partial(
    pl.kernel,
    out_type=jax.ShapeDtypeStruct((NCORE * NSUB, NP), jnp.float32),
    mesh=_mesh,
    compiler_params=_params,
    scratch_types=[
        pltpu.VMEM((EW,), jnp.int32),
        pltpu.VMEM((NP,), jnp.float32),
    ],
)(_deg_body)


_BR = 256


def _prep_body(x_ref, w_ref, degp_ref, p0_ref, dinv_ref):
    h = jnp.dot(x_ref[...], w_ref[...], preferred_element_type=jnp.float32)
    deg = jnp.sum(degp_ref[...], axis=0)
    dinv = jnp.where(deg > 0.0, lax.rsqrt(deg), 0.0)
    p0_ref[...] = h * dinv[:, None]
    dinv_ref[...] = dinv


def _prep_kernel(xp, W, degp):
    return pl.pallas_call(
        _prep_body,
        grid=(NP // _BR,),
        in_specs=[
            pl.BlockSpec((_BR, D), lambda i: (i, 0)),
            pl.BlockSpec((D, C), lambda i: (0, 0)),
            pl.BlockSpec((NCORE * NSUB, _BR), lambda i: (0, i)),
        ],
        out_specs=[
            pl.BlockSpec((_BR, C), lambda i: (i, 0)),
            pl.BlockSpec((_BR,), lambda i: (i,)),
        ],
        out_shape=[
            jax.ShapeDtypeStruct((NP, C), jnp.float32),
            jax.ShapeDtypeStruct((NP,), jnp.float32),
        ],
    )(xp, W, degp)


NBUF = 4


def _prop_body(p0_hbm, row_hbm, col_hbm, dinv_hbm, out_hbm,
               p_sh, s_sh, rows_v, cols_v, gbufs, pbuf, dinv_v, zbuf,
               gsems, ssems):
    cid = lax.axis_index("c")
    sid = lax.axis_index("s")
    rt = sid * RT

    pltpu.sync_copy(row_hbm.at[sid], rows_v)
    pltpu.sync_copy(col_hbm.at[sid], cols_v)
    pltpu.sync_copy(dinv_hbm.at[pl.ds(rt, RT)], dinv_v)

    def zero_body(i, carry):
        zbuf[i, pl.ds(0, HF)] = jnp.zeros((HF,), jnp.float32)
        return carry

    lax.fori_loop(0, RT, zero_body, 0)

    def gstart(c, b):
        pltpu.async_copy(p_sh.at[rows_v.at[c]], gbufs[b], gsems[b])

    def gwait(c, b):
        pltpu.make_async_copy(p_sh.at[rows_v.at[c]], gbufs[b], gsems[b]).wait()

    def sstart(c, b):
        pltpu.async_copy(gbufs[b], s_sh.at[cols_v.at[c]], ssems[b], add=True)

    def swait(c, b):
        pltpu.make_async_copy(gbufs[b], s_sh.at[cols_v.at[c]], ssems[b]).wait()

    for half in range(2):
        pltpu.sync_copy(p0_hbm.at[cid, half, pl.ds(rt, RT)],
                        p_sh.at[pl.ds(rt, RT)])
        pltpu.sync_copy(zbuf, s_sh.at[pl.ds(rt, RT)])
        plsc.subcore_barrier()

        for layer in range(L):
            gstart(0, 0)
            gstart(1, 1)

            def edge_body(j, carry):
                for b in range(NBUF):
                    c = NBUF * j + b
                    bp = (b + 2) % NBUF
                    if b < 2:
                        @pl.when(j > 0)
                        def _():
                            swait(c - 2, bp)
                    else:
                        swait(c - 2, bp)

                    @pl.when(c + 2 < NCHUNK)
                    def _():
                        gstart(c + 2, bp)

                    gwait(c, b)
                    sstart(c, b)
                return carry

            lax.fori_loop(0, NCHUNK // NBUF, edge_body, 0)
            clast = NCHUNK - 1
            gwait(clast, clast % NBUF)
            sstart(clast, clast % NBUF)
            swait(clast - 2, (clast - 2) % NBUF)
            swait(clast - 1, (clast - 1) % NBUF)
            swait(clast, clast % NBUF)
            plsc.subcore_barrier()

            pltpu.sync_copy(s_sh.at[pl.ds(rt, RT)], pbuf)
            last = layer == L - 1

            def scale_body(i, carry):
                d = plsc.load_gather(dinv_v, [jnp.full((16,), i, jnp.int32)])
                f = d if last else d * d
                pbuf[i, pl.ds(0, HF)] = pbuf[i, pl.ds(0, HF)] * f
                return carry

            lax.fori_loop(0, RT, scale_body, 0)

            if last:
                pltpu.sync_copy(pbuf, out_hbm.at[cid, half, pl.ds(rt, RT)])
            else:
                pltpu.sync_copy(pbuf, p_sh.at[pl.ds(rt, RT)])
                pltpu.sync_copy(zbuf, s_sh.at[pl.ds(rt, RT)])
                plsc.subcore_barrier()


_prop_kernel = functools.partial(
    pl.kernel,
    out_type=jax.ShapeDtypeStruct((NCORE, 2, NP, HF), jnp.float32),
    mesh=_mesh,
    compiler_params=_params,
    scratch_types=[
        pltpu.VMEM_SHARED((NP, HF), jnp.float32),
        pltpu.VMEM_SHARED((NP, HF), jnp.float32),
        pltpu.VMEM((NCHUNK, K), jnp.int32),
        pltpu.VMEM((NCHUNK, K), jnp.int32),
        tuple(pltpu.VMEM((K, HF), jnp.float32) for _ in range(NBUF)),
        pltpu.VMEM((RT, HF), jnp.float32),
        pltpu.VMEM((RT,), jnp.float32),
        pltpu.VMEM((RT, HF), jnp.float32),
        tuple(pltpu.SemaphoreType.DMA for _ in range(NBUF)),
        tuple(pltpu.SemaphoreType.DMA for _ in range(NBUF)),
    ],
)(_prop_body)


def kernel(x, edge_index, W):
    row = edge_index[0]
    col = edge_index[1]

    degp = _deg_kernel(row.reshape(NCORE * NSUB, EW))
    xp = jnp.pad(x, ((0, NP - N), (0, 0)))
    p0, dinv = _prep_kernel(xp, W, degp)
    p0q = jnp.stack([
        jnp.stack([p0[:, 0:16], p0[:, 16:32]]),
        jnp.stack([p0[:, 32:48], p0[:, 48:64]]),
    ])
    epad = jnp.full((2, EPAD - E), NP - 1, dtype=jnp.int32)
    eip = jnp.concatenate([edge_index, epad], axis=1)
    out = _prop_kernel(
        p0q,
        eip[0].reshape(NSUB, NCHUNK, K),
        eip[1].reshape(NSUB, NCHUNK, K),
        dinv,
    )
    return jnp.concatenate(
        [out[0, 0], out[0, 1], out[1, 0], out[1, 1]], axis=-1)[:N]


# speedup vs baseline: 1.1499x; 1.1499x over previous
import functools

import jax
import jax.numpy as jnp
from jax import lax
from jax.experimental import pallas as pl
from jax.experimental.pallas import tpu as pltpu
from jax.experimental.pallas import tpu_sc as plsc

N = 10000
E = 320000
D = 128
C = 64
L = 4

NCORE = 2
NSUB = 16
NP = 10240
RT = NP // NSUB
CH = C // NCORE
K = 128
NCHUNK = 157
EPAD = NSUB * NCHUNK * K
EW = E // (NCORE * NSUB)

_mesh = plsc.VectorSubcoreMesh(core_axis_name="c", subcore_axis_name="s")
_params = pltpu.CompilerParams(
    needs_layout_passes=False, use_tc_tiling_on_sc=False)


def _deg_body(row_hbm, out_hbm, idx_v, hist_v):
    cid = lax.axis_index("c")
    sid = lax.axis_index("s")
    wid = cid * NSUB + sid
    pltpu.sync_copy(row_hbm.at[wid], idx_v)

    def zero_body(i, carry):
        hist_v[pl.ds(i * 16, 16)] = jnp.zeros((16,), jnp.float32)
        return carry

    lax.fori_loop(0, NP // 16, zero_body, 0)

    ones16 = jnp.ones((16,), jnp.float32)

    def acc_body(j, carry):
        idx16 = idx_v[pl.ds(j * 16, 16)]
        plsc.addupdate_scatter(hist_v, [idx16], ones16)
        return carry

    lax.fori_loop(0, EW // 16, acc_body, 0)
    pltpu.sync_copy(hist_v, out_hbm.at[wid])


_deg_kernel = functools.partial(
    pl.kernel,
    out_type=jax.ShapeDtypeStruct((NCORE * NSUB, NP), jnp.float32),
    mesh=_mesh,
    compiler_params=_params,
    scratch_types=[
        pltpu.VMEM((EW,), jnp.int32),
        pltpu.VMEM((NP,), jnp.float32),
    ],
)(_deg_body)


_BR = 256


def _prep_body(x_ref, w_ref, degp_ref, p0_ref, dinv_ref):
    h = jnp.dot(x_ref[...], w_ref[...], preferred_element_type=jnp.float32)
    deg = jnp.sum(degp_ref[...], axis=0)
    dinv = jnp.where(deg > 0.0, lax.rsqrt(deg), 0.0)
    p0 = h * dinv[:, None]
    p0_ref[0] = p0[:, :CH]
    p0_ref[1] = p0[:, CH:]
    dinv_ref[...] = dinv


def _prep_kernel(xp, W, degp):
    return pl.pallas_call(
        _prep_body,
        grid=(NP // _BR,),
        in_specs=[
            pl.BlockSpec((_BR, D), lambda i: (i, 0)),
            pl.BlockSpec((D, C), lambda i: (0, 0)),
            pl.BlockSpec((NCORE * NSUB, _BR), lambda i: (0, i)),
        ],
        out_specs=[
            pl.BlockSpec((NCORE, _BR, CH), lambda i: (0, i, 0)),
            pl.BlockSpec((_BR,), lambda i: (i,)),
        ],
        out_shape=[
            jax.ShapeDtypeStruct((NCORE, NP, CH), jnp.float32),
            jax.ShapeDtypeStruct((NP,), jnp.float32),
        ],
    )(xp, W, degp)


def _prop_body(p0_hbm, row_hbm, col_hbm, dinv_hbm, out_hbm,
               p_sh, s_sh, rows_v, cols_v, gbuf0, gbuf1, pbuf, dinv_v, zbuf,
               sem0, sem1):
    cid = lax.axis_index("c")
    sid = lax.axis_index("s")
    rt = sid * RT

    pltpu.sync_copy(row_hbm.at[sid], rows_v)
    pltpu.sync_copy(col_hbm.at[sid], cols_v)
    pltpu.sync_copy(p0_hbm.at[cid, pl.ds(rt, RT)], p_sh.at[pl.ds(rt, RT)])
    pltpu.sync_copy(dinv_hbm.at[pl.ds(rt, RT)], dinv_v)

    def zero_body(i, carry):
        z = jnp.zeros((16,), jnp.float32)
        zbuf[i, pl.ds(0, 16)] = z
        zbuf[i, pl.ds(16, 16)] = z
        return carry

    lax.fori_loop(0, RT, zero_body, 0)
    pltpu.sync_copy(zbuf, s_sh.at[pl.ds(rt, RT)])
    plsc.subcore_barrier()

    for layer in range(L):
        pltpu.async_copy(p_sh.at[rows_v.at[0]], gbuf0, sem0)

        def edge_body(j, carry):
            c0 = 2 * j
            c1 = c0 + 1
            pltpu.async_copy(p_sh.at[rows_v.at[c1]], gbuf1, sem1)
            pltpu.make_async_copy(p_sh.at[rows_v.at[c0]], gbuf0, sem0).wait()
            pltpu.sync_copy(gbuf0, s_sh.at[cols_v.at[c0]], add=True)
            pltpu.async_copy(p_sh.at[rows_v.at[c0 + 2]], gbuf0, sem0)
            pltpu.make_async_copy(p_sh.at[rows_v.at[c1]], gbuf1, sem1).wait()
            pltpu.sync_copy(gbuf1, s_sh.at[cols_v.at[c1]], add=True)
            return carry

        lax.fori_loop(0, NCHUNK // 2, edge_body, 0)
        clast = NCHUNK - 1
        pltpu.make_async_copy(p_sh.at[rows_v.at[clast]], gbuf0, sem0).wait()
        pltpu.sync_copy(gbuf0, s_sh.at[cols_v.at[clast]], add=True)
        plsc.subcore_barrier()

        pltpu.sync_copy(s_sh.at[pl.ds(rt, RT)], pbuf)
        last = layer == L - 1

        def scale_body(i, carry):
            d = plsc.load_gather(dinv_v, [jnp.full((16,), i, jnp.int32)])
            f = d if last else d * d
            pbuf[i, pl.ds(0, 16)] = pbuf[i, pl.ds(0, 16)] * f
            pbuf[i, pl.ds(16, 16)] = pbuf[i, pl.ds(16, 16)] * f
            return carry

        lax.fori_loop(0, RT, scale_body, 0)

        if last:
            pltpu.sync_copy(pbuf, out_hbm.at[pl.ds(rt, RT), pl.ds(cid * CH, CH)])
        else:
            pltpu.sync_copy(pbuf, p_sh.at[pl.ds(rt, RT)])
            pltpu.sync_copy(zbuf, s_sh.at[pl.ds(rt, RT)])
            plsc.subcore_barrier()


_prop_kernel = functools.partial(
    pl.kernel,
    out_type=jax.ShapeDtypeStruct((NP, C), jnp.float32),
    mesh=_mesh,
    compiler_params=_params,
    scratch_types=[
        pltpu.VMEM_SHARED((NP, CH), jnp.float32),
        pltpu.VMEM_SHARED((NP, CH), jnp.float32),
        pltpu.VMEM((NCHUNK, K), jnp.int32),
        pltpu.VMEM((NCHUNK, K), jnp.int32),
        pltpu.VMEM((K, CH), jnp.float32),
        pltpu.VMEM((K, CH), jnp.float32),
        pltpu.VMEM((RT, CH), jnp.float32),
        pltpu.VMEM((RT,), jnp.float32),
        pltpu.VMEM((RT, CH), jnp.float32),
        pltpu.SemaphoreType.DMA,
        pltpu.SemaphoreType.DMA,
    ],
)(_prop_body)


def kernel(x, edge_index, W):
    row = edge_index[0]
    col = edge_index[1]

    degp = _deg_kernel(row.reshape(NCORE * NSUB, EW))
    xp = jnp.pad(x, ((0, NP - N), (0, 0)))
    p0s, dinv = _prep_kernel(xp, W, degp)
    epad = jnp.full((2, EPAD - E), NP - 1, dtype=jnp.int32)
    eip = jnp.concatenate([edge_index, epad], axis=1)
    out = _prop_kernel(
        p0s,
        eip[0].reshape(NSUB, NCHUNK, K),
        eip[1].reshape(NSUB, NCHUNK, K),
        dinv,
    )
    return out[:N]


# speedup vs baseline: 1.1575x; 1.0066x over previous
import functools

import jax
import jax.numpy as jnp
from jax import lax
from jax.experimental import pallas as pl
from jax.experimental.pallas import tpu as pltpu
from jax.experimental.pallas import tpu_sc as plsc

N = 10000
E = 320000
D = 128
C = 64
L = 4

NCORE = 2
NSUB = 16
NP = 10240
RT = NP // NSUB
CH = C // NCORE
K = 128
NCHUNK = 157
EPAD = NSUB * NCHUNK * K
EW = E // (NCORE * NSUB)

_mesh = plsc.VectorSubcoreMesh(core_axis_name="c", subcore_axis_name="s")
_params = pltpu.CompilerParams(
    needs_layout_passes=False, use_tc_tiling_on_sc=False)


def _deg_body(row_hbm, out_hbm, idx_v, hist_v):
    cid = lax.axis_index("c")
    sid = lax.axis_index("s")
    wid = cid * NSUB + sid
    pltpu.sync_copy(row_hbm.at[wid], idx_v)

    def zero_body(i, carry):
        hist_v[pl.ds(i * 16, 16)] = jnp.zeros((16,), jnp.float32)
        return carry

    lax.fori_loop(0, NP // 16, zero_body, 0)

    ones16 = jnp.ones((16,), jnp.float32)

    def acc_body(j, carry):
        idx16 = idx_v[pl.ds(j * 16, 16)]
        plsc.addupdate_scatter(hist_v, [idx16], ones16)
        return carry

    lax.fori_loop(0, EW // 16, acc_body, 0)
    pltpu.sync_copy(hist_v, out_hbm.at[wid])


_deg_kernel = functools.partial(
    pl.kernel,
    out_type=jax.ShapeDtypeStruct((NCORE * NSUB, NP), jnp.float32),
    mesh=_mesh,
    compiler_params=_params,
    scratch_types=[
        pltpu.VMEM((EW,), jnp.int32),
        pltpu.VMEM((NP,), jnp.float32),
    ],
)(_deg_body)


_BR = 256


def _prep_body(x_ref, w_ref, degp_ref, p0_ref, dinv_ref):
    h = jnp.dot(x_ref[...], w_ref[...], preferred_element_type=jnp.float32)
    deg = jnp.sum(degp_ref[...], axis=0)
    dinv = jnp.where(deg > 0.0, lax.rsqrt(deg), 0.0)
    p0 = h * dinv[:, None]
    p0_ref[0] = p0[:, :CH]
    p0_ref[1] = p0[:, CH:]
    dinv_ref[...] = dinv


def _prep_kernel(xp, W, degp):
    return pl.pallas_call(
        _prep_body,
        grid=(NP // _BR,),
        in_specs=[
            pl.BlockSpec((_BR, D), lambda i: (i, 0)),
            pl.BlockSpec((D, C), lambda i: (0, 0)),
            pl.BlockSpec((NCORE * NSUB, _BR), lambda i: (0, i)),
        ],
        out_specs=[
            pl.BlockSpec((NCORE, _BR, CH), lambda i: (0, i, 0)),
            pl.BlockSpec((_BR,), lambda i: (i,)),
        ],
        out_shape=[
            jax.ShapeDtypeStruct((NCORE, NP, CH), jnp.float32),
            jax.ShapeDtypeStruct((NP,), jnp.float32),
        ],
    )(xp, W, degp)


def _prop_body(p0_hbm, row_hbm, col_hbm, dinv_hbm, out_hbm,
               p_sh, s_sh, rows_v, cols_v, gbuf0, gbuf1, pbuf, dinv_v, zbuf,
               sem0, sem1):
    cid = lax.axis_index("c")
    sid = lax.axis_index("s")
    rt = sid * RT

    pltpu.async_copy(row_hbm.at[sid], rows_v, sem0)
    pltpu.async_copy(col_hbm.at[sid], cols_v, sem1)
    pltpu.sync_copy(p0_hbm.at[cid, pl.ds(rt, RT)], p_sh.at[pl.ds(rt, RT)])
    pltpu.sync_copy(dinv_hbm.at[pl.ds(rt, RT)], dinv_v)

    def zero_body(i, carry):
        z = jnp.zeros((16,), jnp.float32)
        zbuf[i, pl.ds(0, 16)] = z
        zbuf[i, pl.ds(16, 16)] = z
        return carry

    lax.fori_loop(0, RT, zero_body, 0)
    pltpu.sync_copy(zbuf, s_sh.at[pl.ds(rt, RT)])
    pltpu.make_async_copy(row_hbm.at[sid], rows_v, sem0).wait()
    pltpu.make_async_copy(col_hbm.at[sid], cols_v, sem1).wait()
    plsc.subcore_barrier()

    for layer in range(L):
        pltpu.async_copy(p_sh.at[rows_v.at[0]], gbuf0, sem0)

        def edge_body(j, carry):
            c0 = 2 * j
            c1 = c0 + 1
            pltpu.async_copy(p_sh.at[rows_v.at[c1]], gbuf1, sem1)
            pltpu.make_async_copy(p_sh.at[rows_v.at[c0]], gbuf0, sem0).wait()
            pltpu.sync_copy(gbuf0, s_sh.at[cols_v.at[c0]], add=True)
            pltpu.async_copy(p_sh.at[rows_v.at[c0 + 2]], gbuf0, sem0)
            pltpu.make_async_copy(p_sh.at[rows_v.at[c1]], gbuf1, sem1).wait()
            pltpu.sync_copy(gbuf1, s_sh.at[cols_v.at[c1]], add=True)
            return carry

        lax.fori_loop(0, NCHUNK // 2, edge_body, 0)
        clast = NCHUNK - 1
        pltpu.make_async_copy(p_sh.at[rows_v.at[clast]], gbuf0, sem0).wait()
        pltpu.sync_copy(gbuf0, s_sh.at[cols_v.at[clast]], add=True)
        plsc.subcore_barrier()

        pltpu.sync_copy(s_sh.at[pl.ds(rt, RT)], pbuf)
        last = layer == L - 1

        def scale_body(i, carry):
            d = plsc.load_gather(dinv_v, [jnp.full((16,), i, jnp.int32)])
            f = d if last else d * d
            pbuf[i, pl.ds(0, 16)] = pbuf[i, pl.ds(0, 16)] * f
            pbuf[i, pl.ds(16, 16)] = pbuf[i, pl.ds(16, 16)] * f
            return carry

        lax.fori_loop(0, RT, scale_body, 0)

        if last:
            pltpu.sync_copy(pbuf, out_hbm.at[pl.ds(rt, RT), pl.ds(cid * CH, CH)])
        else:
            pltpu.sync_copy(pbuf, p_sh.at[pl.ds(rt, RT)])
            pltpu.sync_copy(zbuf, s_sh.at[pl.ds(rt, RT)])
            plsc.subcore_barrier()


_prop_kernel = functools.partial(
    pl.kernel,
    out_type=jax.ShapeDtypeStruct((NP, C), jnp.float32),
    mesh=_mesh,
    compiler_params=_params,
    scratch_types=[
        pltpu.VMEM_SHARED((NP, CH), jnp.float32),
        pltpu.VMEM_SHARED((NP, CH), jnp.float32),
        pltpu.VMEM((NCHUNK, K), jnp.int32),
        pltpu.VMEM((NCHUNK, K), jnp.int32),
        pltpu.VMEM((K, CH), jnp.float32),
        pltpu.VMEM((K, CH), jnp.float32),
        pltpu.VMEM((RT, CH), jnp.float32),
        pltpu.VMEM((RT,), jnp.float32),
        pltpu.VMEM((RT, CH), jnp.float32),
        pltpu.SemaphoreType.DMA,
        pltpu.SemaphoreType.DMA,
    ],
)(_prop_body)


def kernel(x, edge_index, W):
    row = edge_index[0]
    col = edge_index[1]

    degp = _deg_kernel(row.reshape(NCORE * NSUB, EW))
    xp = jnp.pad(x, ((0, NP - N), (0, 0)))
    p0s, dinv = _prep_kernel(xp, W, degp)
    epad = jnp.full((2, EPAD - E), NP - 1, dtype=jnp.int32)
    eip = jnp.concatenate([edge_index, epad], axis=1)
    out = _prop_kernel(
        p0s,
        eip[0].reshape(NSUB, NCHUNK, K),
        eip[1].reshape(NSUB, NCHUNK, K),
        dinv,
    )
    return out[:N]
